# Initial kernel scaffold; baseline (speedup 1.0000x reference)
import jax
import jax.numpy as jnp
from jax.experimental import pallas as pl


def kernel(features, rois):
    raise NotImplementedError("write your pallas kernel here")


# speedup vs baseline: 3.0193x; 3.0193x over previous
import functools

import jax
import jax.numpy as jnp
from jax import lax
from jax.experimental import pallas as pl
from jax.experimental.pallas import tpu as pltpu
from jax.experimental.pallas import tpu_sc as plsc

_SCALE = 0.0625
_AH = 7
_AW = 7
_NS = _AH * _AW
_B, _C, _H, _W = 2, 256, 64, 64
_NROI = 2000
_NWORKERS = 32
_ROWS = 2 * _NS
_ROWS_PAD = 104
_PAIRW = 2 * _C
_OUTW = _C * _NS


def _roi_align_body(ptab, roisp, out, roibuf, idxbuf, wbuf, gbuf, outbuf, sem):
    cid = lax.axis_index("c")
    sid = lax.axis_index("s")
    wid = sid * 2 + cid
    start = wid * 62 + jnp.minimum(wid, 16)
    pltpu.sync_copy(roisp.at[pl.ds(start * 16, 64 * 16)], roibuf)

    iota = lax.iota(jnp.int32, 16)
    iota_ns = iota * _NS
    idxbuf[pl.ds(_ROWS_PAD - 16, 16)] = jnp.zeros((16,), jnp.int32)

    def _splat_load(ref, i):
        return plsc.load_gather(ref, [jnp.broadcast_to(i, (16,))])

    def roi_body(j, carry):
        n = jnp.minimum(start + j, _NROI - 1)
        local = (n - start) * 16
        bv = _splat_load(roibuf, local).astype(jnp.int32)
        x1 = _splat_load(roibuf, local + 1) * _SCALE
        y1 = _splat_load(roibuf, local + 2) * _SCALE
        x2 = _splat_load(roibuf, local + 3) * _SCALE
        y2 = _splat_load(roibuf, local + 4) * _SCALE
        binh = jnp.maximum(y2 - y1 + 1.0, 0.0) * (1.0 / (_AH - 1))
        binw = jnp.maximum(x2 - x1 + 1.0, 0.0) * (1.0 / (_AW - 1))

        for r in range(4):
            s = iota + 16 * r
            ph = (s // _AW).astype(jnp.float32)
            pw = (s % _AW).astype(jnp.float32)
            hs = y1 + ph * binh
            ws = x1 + pw * binw
            valid = (hs >= 0.0) & (hs < float(_H)) & (ws >= 0.0) & (ws < float(_W))
            hst = jnp.clip(hs.astype(jnp.int32), 0, _H - 2)
            wst = jnp.clip(ws.astype(jnp.int32), 0, _W - 2)
            hr = hs - hst.astype(jnp.float32)
            wr = ws - wst.astype(jnp.float32)
            vf = jnp.where(valid, 1.0, 0.0)
            omh = (1.0 - hr) * vf
            hrv = hr * vf
            wbuf[pl.ds(16 * r, 16)] = omh * (1.0 - wr)
            wbuf[pl.ds(64 + 16 * r, 16)] = omh * wr
            wbuf[pl.ds(128 + 16 * r, 16)] = hrv * (1.0 - wr)
            wbuf[pl.ds(192 + 16 * r, 16)] = hrv * wr
            idx = bv * (_H * _W) + hst * _W + wst
            m = s < _NS
            plsc.store_scatter(idxbuf, [s], idx, mask=m)
            plsc.store_scatter(idxbuf, [s + _NS], idx + 1, mask=m)

        pltpu.async_copy(ptab.at[idxbuf], gbuf, sem).wait()

        def s_body(s, c2):
            w0 = _splat_load(wbuf, s)
            w1 = _splat_load(wbuf, s + 64)
            w2 = _splat_load(wbuf, s + 128)
            w3 = _splat_load(wbuf, s + 192)
            base = iota_ns + s
            for k in range(_C // 16):
                ul = gbuf[s, pl.ds(16 * k, 16)]
                ll = gbuf[s, pl.ds(_C + 16 * k, 16)]
                ur = gbuf[s + _NS, pl.ds(16 * k, 16)]
                lr = gbuf[s + _NS, pl.ds(_C + 16 * k, 16)]
                acc = ul * w0 + ur * w1 + ll * w2 + lr * w3
                plsc.store_scatter(outbuf, [base + (16 * _NS) * k], acc)
            return c2

        lax.fori_loop(0, _NS, s_body, 0)
        pltpu.sync_copy(outbuf, out.at[pl.ds(n * _OUTW, _OUTW)])
        return carry

    lax.fori_loop(0, 63, roi_body, 0)


_roi_align_sc = functools.partial(
    pl.kernel,
    out_type=jax.ShapeDtypeStruct((_NROI * _OUTW,), jnp.float32),
    mesh=plsc.VectorSubcoreMesh(core_axis_name="c", subcore_axis_name="s"),
    compiler_params=pltpu.CompilerParams(needs_layout_passes=False),
    scratch_types=[
        pltpu.VMEM((64 * 16,), jnp.float32),
        pltpu.VMEM((_ROWS_PAD,), jnp.int32),
        pltpu.VMEM((4 * 64,), jnp.float32),
        pltpu.VMEM((_ROWS_PAD, _PAIRW), jnp.float32),
        pltpu.VMEM((_OUTW,), jnp.float32),
        pltpu.SemaphoreType.DMA,
    ],
)(_roi_align_body)


def kernel(features, rois):
    B, C, H, W = features.shape
    n = rois.shape[0]
    flat = jnp.transpose(features, (0, 2, 3, 1)).reshape(B * H * W, C)
    ptab = jnp.concatenate([flat[: B * H * W - W], flat[W:]], axis=1)
    roisp = jnp.zeros((2048, 16), jnp.float32).at[:n, :5].set(rois).reshape(-1)
    out = _roi_align_sc(ptab, roisp)
    return out.reshape(n, C, _AH, _AW)
